# baseline (speedup 1.0000x reference)
import dataclasses
import functools

import jax
import jax.numpy as jnp
from jax.experimental import pallas as pl
from jax.experimental.pallas import tpu as pltpu
from jax.experimental.pallas import tpu_sc as plsc

N = 4096
D = 128
D_OUT = 64
K_IDX = 16609443

BM = 512
NB = N // BM

NW = 32
LANES = 16
H1_R, H1_C = 136, 128
HIST1 = H1_R * H1_C
H2_R, H2_C = 512, 128
HIST2 = H2_R * H2_C

_SC_CP = pltpu.CompilerParams()
if "needs_layout_passes" in pltpu.CompilerParams.__dataclass_fields__:
    _SC_CP = dataclasses.replace(_SC_CP, needs_layout_passes=False)


def _prep_kernel(x_ref, w_in_ref, xn_ref, h_ref):
    x = x_ref[...]
    nrm = jnp.sqrt(jnp.sum(x * x, axis=1, keepdims=True))
    xn_ref[...] = x / jnp.maximum(nrm, 1e-8)
    h = jax.lax.dot_general(
        x, w_in_ref[...], (((1,), (1,)), ((), ())),
        preferred_element_type=jnp.float32,
    )
    h_ref[...] = jnp.maximum(h, 0.0)


def _sim_kernel(xn_blk_ref, xn_ref, bits_ref):
    s = jax.lax.dot_general(
        xn_blk_ref[...], xn_ref[...], (((1,), (1,)), ((), ())),
        preferred_element_type=jnp.float32,
    )
    bits_ref[...] = pltpu.bitcast(jnp.abs(s), jnp.int32)


def _sc_hist1_kernel(bits_hbm, out_hbm, hist_v, sem):
    wid = jax.lax.axis_index("s") * 2 + jax.lax.axis_index("c")

    @pl.loop(0, HIST1, step=LANES)
    def _zero(i):
        hist_v.at[pl.ds(i, LANES)][...] = jnp.zeros((LANES,), jnp.int32)

    ones = jnp.ones((LANES,), jnp.int32)

    def body(in_v):
        @pl.loop(0, N, step=4 * LANES)
        def _(c):
            for u in range(4):
                v = in_v.at[0, pl.ds(c + u * LANES, LANES)][...]
                b = jax.lax.shift_right_logical(v, 16)
                plsc.addupdate_scatter(hist_v, [b], ones)

    pltpu.emit_pipeline(
        body,
        grid=(N,),
        in_specs=[pl.BlockSpec((1, N), lambda i: (i, 0))],
        out_specs=[],
        core_axis_name=("c", "s"),
        dimension_semantics=(pltpu.PARALLEL,),
    )(bits_hbm)
    pltpu.async_copy(hist_v, out_hbm.at[wid], sem).wait()


def _sc_hist2_kernel(bits_hbm, bsel_hbm, out_hbm, hist_v, bsel_v, sem):
    wid = jax.lax.axis_index("s") * 2 + jax.lax.axis_index("c")
    pltpu.async_copy(bsel_hbm, bsel_v, sem).wait()

    @pl.loop(0, HIST2, step=LANES)
    def _zero(i):
        hist_v.at[pl.ds(i, LANES)][...] = jnp.zeros((LANES,), jnp.int32)

    ones = jnp.ones((LANES,), jnp.int32)
    bvec = bsel_v.at[0][...]
    lowmask = jnp.full((LANES,), 0xFFFF, jnp.int32)

    def body(in_v):
        @pl.loop(0, N, step=4 * LANES)
        def _(c):
            for u in range(4):
                v = in_v.at[0, pl.ds(c + u * LANES, LANES)][...]
                hi = jax.lax.shift_right_logical(v, 16)
                b2 = jnp.bitwise_and(v, lowmask)
                plsc.addupdate_scatter(hist_v, [b2], ones, mask=hi == bvec)

    pltpu.emit_pipeline(
        body,
        grid=(N,),
        in_specs=[pl.BlockSpec((1, N), lambda i: (i, 0))],
        out_specs=[],
        core_axis_name=("c", "s"),
        dimension_semantics=(pltpu.PARALLEL,),
    )(bits_hbm)
    pltpu.async_copy(hist_v, out_hbm.at[wid], sem).wait()


def _rowmajor_cum(hs, rows, cols):
    ric = jax.lax.broadcasted_iota(jnp.int32, (cols, cols), 0)
    cic = jax.lax.broadcasted_iota(jnp.int32, (cols, cols), 1)
    ut = (ric <= cic).astype(jnp.float32)
    cum_row = jax.lax.dot_general(
        hs, ut, (((1,), (0,)), ((), ())), preferred_element_type=jnp.float32)
    tot = cum_row[:, cols - 1:cols]
    rir = jax.lax.broadcasted_iota(jnp.int32, (rows, rows), 0)
    cir = jax.lax.broadcasted_iota(jnp.int32, (rows, rows), 1)
    lt = (cir < rir).astype(jnp.float32)
    prev = jax.lax.dot_general(
        lt, tot, (((1,), (0,)), ((), ())), preferred_element_type=jnp.float32)
    return cum_row + prev


def _pick1_kernel(h1_ref, out_ref):
    K1 = float(K_IDX + 1)
    hs = jnp.sum(h1_ref[...].astype(jnp.float32), axis=0)
    cum = _rowmajor_cum(hs, H1_R, H1_C)
    lin = (jax.lax.broadcasted_iota(jnp.int32, (H1_R, H1_C), 0) * H1_C
           + jax.lax.broadcasted_iota(jnp.int32, (H1_R, H1_C), 1))
    mask = cum >= K1
    big = jnp.int32(2 ** 30)
    bsel = jnp.min(jnp.where(mask, lin, big))
    cum_at_b = jnp.min(jnp.where(mask, cum, jnp.float32(3e8)))
    hist_at_b = jnp.sum(jnp.where(lin == bsel, hs, 0.0))
    count_below = cum_at_b - hist_at_b
    r_rank = jnp.int32(K_IDX) - count_below.astype(jnp.int32)
    rowi = jax.lax.broadcasted_iota(jnp.int32, (2, LANES), 0)
    out_ref[...] = jnp.where(rowi == 0, bsel, r_rank)


def _pick2_kernel(h2_ref, bsel_ref, eps_ref):
    hs = jnp.sum(h2_ref[...].astype(jnp.float32), axis=0)
    cum = _rowmajor_cum(hs, H2_R, H2_C)
    r1 = bsel_ref[1, 0].astype(jnp.float32) + 1.0
    lin = (jax.lax.broadcasted_iota(jnp.int32, (H2_R, H2_C), 0) * H2_C
           + jax.lax.broadcasted_iota(jnp.int32, (H2_R, H2_C), 1))
    low = jnp.min(jnp.where(cum >= r1, lin, jnp.int32(2 ** 30)))
    eps_ref[0, 0] = jnp.left_shift(bsel_ref[0, 0], 16) | low


def _sage_kernel(eps_ref, bits_ref, h_ref, h_blk_ref, wl_ref, bl_ref,
                 wr_ref, wo_ref, bo_ref, out_ref):
    eps = eps_ref[0]
    mask = (bits_ref[...] >= eps).astype(jnp.float32)
    deg = jnp.sum(mask, axis=1, keepdims=True)
    aggn = jnp.dot(mask, h_ref[...], preferred_element_type=jnp.float32)
    agg = aggn / jnp.maximum(deg, 1.0)
    z = (
        jax.lax.dot_general(
            agg, wl_ref[...], (((1,), (1,)), ((), ())),
            preferred_element_type=jnp.float32,
        )
        + bl_ref[...]
        + jax.lax.dot_general(
            h_blk_ref[...], wr_ref[...], (((1,), (1,)), ((), ())),
            preferred_element_type=jnp.float32,
        )
    )
    h2 = jnp.maximum(z, 0.0)
    o = jax.lax.dot_general(
        h2, wo_ref[...], (((1,), (1,)), ((), ())),
        preferred_element_type=jnp.float32,
    ) + bo_ref[...]
    out_ref[...] = jax.nn.sigmoid(o)


@jax.jit
def kernel(x, W_in, W_l, b_l, W_r, W_out, b_out):
    xn, h = pl.pallas_call(
        _prep_kernel,
        out_shape=(
            jax.ShapeDtypeStruct((N, D), jnp.float32),
            jax.ShapeDtypeStruct((N, D), jnp.float32),
        ),
    )(x, W_in)

    bits = pl.pallas_call(
        _sim_kernel,
        grid=(NB,),
        in_specs=[
            pl.BlockSpec((BM, D), lambda i: (i, 0)),
            pl.BlockSpec((N, D), lambda i: (0, 0)),
        ],
        out_specs=pl.BlockSpec((BM, N), lambda i: (i, 0)),
        out_shape=jax.ShapeDtypeStruct((N, N), jnp.int32),
    )(xn, xn)

    mesh = plsc.VectorSubcoreMesh(core_axis_name="c", subcore_axis_name="s")

    h1 = functools.partial(
        pl.kernel,
        out_type=jax.ShapeDtypeStruct((NW, HIST1), jnp.int32),
        mesh=mesh,
        compiler_params=_SC_CP,
        scratch_types=[
            pltpu.VMEM((HIST1,), jnp.int32),
            pltpu.SemaphoreType.DMA,
        ],
    )(_sc_hist1_kernel)(bits)

    bsel = pl.pallas_call(
        _pick1_kernel,
        in_specs=[pl.BlockSpec((NW, H1_R, H1_C), lambda: (0, 0, 0))],
        out_shape=jax.ShapeDtypeStruct((2, LANES), jnp.int32),
    )(h1.reshape(NW, H1_R, H1_C))

    h2 = functools.partial(
        pl.kernel,
        out_type=jax.ShapeDtypeStruct((NW, HIST2), jnp.int32),
        mesh=mesh,
        compiler_params=_SC_CP,
        scratch_types=[
            pltpu.VMEM((HIST2,), jnp.int32),
            pltpu.VMEM((2, LANES), jnp.int32),
            pltpu.SemaphoreType.DMA,
        ],
    )(_sc_hist2_kernel)(bits, bsel)

    eps_bits = pl.pallas_call(
        _pick2_kernel,
        in_specs=[
            pl.BlockSpec((NW, H2_R, H2_C), lambda: (0, 0, 0)),
            pl.BlockSpec(memory_space=pltpu.SMEM),
        ],
        out_specs=pl.BlockSpec(memory_space=pltpu.SMEM),
        out_shape=jax.ShapeDtypeStruct((1, 1), jnp.int32),
    )(h2.reshape(NW, H2_R, H2_C), bsel)

    out = pl.pallas_call(
        _sage_kernel,
        grid=(NB,),
        in_specs=[
            pl.BlockSpec(memory_space=pltpu.SMEM),
            pl.BlockSpec((BM, N), lambda i: (i, 0)),
            pl.BlockSpec((N, D), lambda i: (0, 0)),
            pl.BlockSpec((BM, D), lambda i: (i, 0)),
            pl.BlockSpec((D, D), lambda i: (0, 0)),
            pl.BlockSpec((1, D), lambda i: (0, 0)),
            pl.BlockSpec((D, D), lambda i: (0, 0)),
            pl.BlockSpec((D_OUT, D), lambda i: (0, 0)),
            pl.BlockSpec((1, D_OUT), lambda i: (0, 0)),
        ],
        out_specs=pl.BlockSpec((BM, D_OUT), lambda i: (i, 0)),
        out_shape=jax.ShapeDtypeStruct((N, D_OUT), jnp.float32),
    )(
        eps_bits.reshape(-1), bits, h, h,
        W_l, b_l.reshape(1, D), W_r, W_out, b_out.reshape(1, D_OUT),
    )
    return out
